# Initial kernel scaffold; baseline (speedup 1.0000x reference)
import jax
import jax.numpy as jnp
from jax.experimental import pallas as pl


def kernel(sentence, table):
    raise NotImplementedError("write your pallas kernel here")


# speedup vs baseline: 1.0603x; 1.0603x over previous
import jax
import jax.numpy as jnp
from jax import lax
from jax.experimental import pallas as pl
from jax.experimental.pallas import tpu as pltpu
from jax.experimental.pallas import tpu_sc as plsc

_WINDOW = 128
_DPAD = 384


def kernel(sentence, table):
    batch, seq = sentence.shape
    vocab, dim = table.shape
    n = batch * seq
    idx = sentence.reshape(n).astype(jnp.int32)
    tab = jnp.pad(table, ((0, 0), (0, _DPAD - dim)))

    info = plsc.get_sparse_core_info()
    nw = info.num_cores * info.num_subcores
    per_w = n // nw
    steps = per_w // _WINDOW

    mesh = plsc.VectorSubcoreMesh(core_axis_name="core", subcore_axis_name="subcore")

    @pl.kernel(
        out_type=jax.ShapeDtypeStruct((n, _DPAD), table.dtype),
        mesh=mesh,
        scratch_types=[
            pltpu.VMEM((_WINDOW,), jnp.int32),
            pltpu.VMEM((_WINDOW, _DPAD), jnp.float32),
            pltpu.SemaphoreType.DMA,
        ],
    )
    def gather_kernel(tab_hbm, idx_hbm, out_hbm, idx_v, rows_v, sem):
        wid = lax.axis_index("subcore") * info.num_cores + lax.axis_index("core")
        base = wid * per_w

        @pl.loop(0, steps)
        def _(c):
            off = base + c * _WINDOW
            pltpu.sync_copy(idx_hbm.at[pl.ds(off, _WINDOW)], idx_v)
            pltpu.async_copy(tab_hbm.at[idx_v], rows_v, sem).wait()
            pltpu.sync_copy(rows_v, out_hbm.at[pl.ds(off, _WINDOW)])

    out = gather_kernel(tab, idx)
    return out[:, :dim].reshape(batch, seq, dim)


# speedup vs baseline: 1.5769x; 1.4872x over previous
import jax
import jax.numpy as jnp
from jax import lax
from jax.experimental import pallas as pl
from jax.experimental.pallas import tpu as pltpu
from jax.experimental.pallas import tpu_sc as plsc

_WINDOW = 128


def kernel(sentence, table):
    batch, seq = sentence.shape
    vocab, dim = table.shape
    n = batch * seq
    idx = sentence.reshape(n).astype(jnp.int32)

    info = plsc.get_sparse_core_info()
    nw = info.num_cores * info.num_subcores
    per_w = n // nw
    steps = per_w // _WINDOW

    mesh = plsc.VectorSubcoreMesh(core_axis_name="core", subcore_axis_name="subcore")

    @pl.kernel(
        out_type=jax.ShapeDtypeStruct((n, dim), table.dtype),
        mesh=mesh,
        scratch_types=[
            pltpu.VMEM((_WINDOW,), jnp.int32),
            pltpu.VMEM((_WINDOW, dim), jnp.float32),
            pltpu.SemaphoreType.DMA,
        ],
    )
    def gather_kernel(tab_hbm, idx_hbm, out_hbm, idx_v, rows_v, sem):
        wid = lax.axis_index("subcore") * info.num_cores + lax.axis_index("core")
        base = wid * per_w

        @pl.loop(0, steps)
        def _(c):
            off = base + c * _WINDOW
            pltpu.sync_copy(idx_hbm.at[pl.ds(off, _WINDOW)], idx_v)

            @pl.loop(0, _WINDOW)
            def _(j):
                i = idx_v[pl.ds(j, 1)][0]
                pltpu.async_copy(tab_hbm.at[i], rows_v.at[j], sem)

            pltpu.make_async_copy(tab_hbm.at[pl.ds(0, _WINDOW)], rows_v, sem).wait()
            pltpu.sync_copy(rows_v, out_hbm.at[pl.ds(off, _WINDOW)])

    out = gather_kernel(table, idx)
    return out.reshape(batch, seq, dim)


# speedup vs baseline: 1.6799x; 1.0653x over previous
import jax
import jax.numpy as jnp
from jax import lax
from jax.experimental import pallas as pl
from jax.experimental.pallas import tpu as pltpu
from jax.experimental.pallas import tpu_sc as plsc

_WINDOW = 128
_UNROLL = 16


def kernel(sentence, table):
    batch, seq = sentence.shape
    vocab, dim = table.shape
    n = batch * seq
    idx = sentence.reshape(n).astype(jnp.int32)

    info = plsc.get_sparse_core_info()
    nw = info.num_cores * info.num_subcores
    per_w = n // nw
    steps = per_w // _WINDOW
    assert steps % 2 == 0

    mesh = plsc.VectorSubcoreMesh(core_axis_name="core", subcore_axis_name="subcore")

    @pl.kernel(
        out_type=jax.ShapeDtypeStruct((n, dim), table.dtype),
        mesh=mesh,
        scratch_types=[
            pltpu.VMEM((per_w,), jnp.int32),
            pltpu.VMEM((_WINDOW, dim), jnp.float32),
            pltpu.VMEM((_WINDOW, dim), jnp.float32),
            pltpu.SemaphoreType.DMA,
            pltpu.SemaphoreType.DMA,
        ],
    )
    def gather_kernel(tab_hbm, idx_hbm, out_hbm, idx_v, rows_a, rows_b, sem_a, sem_b):
        wid = lax.axis_index("subcore") * info.num_cores + lax.axis_index("core")
        base = wid * per_w
        pltpu.sync_copy(idx_hbm.at[pl.ds(base, per_w)], idx_v)

        def issue(w, rows, sem):
            @pl.loop(0, _WINDOW, step=_UNROLL)
            def _(j):
                v = idx_v[pl.ds(w * _WINDOW + j, _UNROLL)]
                for k in range(_UNROLL):
                    pltpu.async_copy(tab_hbm.at[v[k]], rows.at[j + k], sem)

        def drain_writeback(w, rows, sem):
            pltpu.make_async_copy(tab_hbm.at[pl.ds(0, _WINDOW)], rows, sem).wait()
            pltpu.sync_copy(rows, out_hbm.at[pl.ds(base + w * _WINDOW, _WINDOW)])

        @pl.loop(0, steps, step=2)
        def _(w):
            issue(w, rows_a, sem_a)
            issue(w + 1, rows_b, sem_b)
            drain_writeback(w, rows_a, sem_a)
            drain_writeback(w + 1, rows_b, sem_b)

    out = gather_kernel(table, idx)
    return out.reshape(batch, seq, dim)
